# baseline (speedup 1.0000x reference)
import functools

import jax
import jax.numpy as jnp
from jax import lax
from jax.experimental import pallas as pl
from jax.experimental.pallas import tpu as pltpu
from jax.experimental.pallas import tpu_sc as plsc

BATCH = 16384
EMB = 64
NUM_WORKERS = 32
B_PER_W = BATCH // NUM_WORKERS
CHUNK = 128
N_CHUNKS = B_PER_W // CHUNK


def _gather_kernel(uids_hbm, mids_hbm, uemb_hbm, memb_hbm,
                   u_out, m_out, uidx_v, midx_v, urows_v, mrows_v, sem):
    wid = lax.axis_index("s") * 2 + lax.axis_index("c")
    base = wid * B_PER_W
    pltpu.sync_copy(uids_hbm.at[pl.ds(base, B_PER_W)], uidx_v)
    pltpu.sync_copy(mids_hbm.at[pl.ds(base, B_PER_W)], midx_v)
    copies = []
    for c in range(N_CHUNKS):
        sl = pl.ds(c * CHUNK, CHUNK)
        copies.append(pltpu.async_copy(
            uemb_hbm.at[uidx_v.at[sl]], urows_v.at[sl], sem))
        copies.append(pltpu.async_copy(
            memb_hbm.at[midx_v.at[sl]], mrows_v.at[sl], sem))
    for cp in copies:
        cp.wait()
    pltpu.sync_copy(urows_v, u_out.at[pl.ds(base, B_PER_W)])
    pltpu.sync_copy(mrows_v, m_out.at[pl.ds(base, B_PER_W)])


def _sc_gather(user_ids, movie_ids, user_emb, movie_emb):
    mesh = plsc.VectorSubcoreMesh(core_axis_name="c", subcore_axis_name="s")
    row_t = jax.ShapeDtypeStruct((BATCH, EMB), jnp.float32)
    k = pl.kernel(
        _gather_kernel,
        out_type=(row_t, row_t),
        mesh=mesh,
        compiler_params=pltpu.CompilerParams(use_tc_tiling_on_sc=False),
        scratch_types=[
            pltpu.VMEM((B_PER_W,), jnp.int32),
            pltpu.VMEM((B_PER_W,), jnp.int32),
            pltpu.VMEM((B_PER_W, EMB), jnp.float32),
            pltpu.VMEM((B_PER_W, EMB), jnp.float32),
            pltpu.SemaphoreType.DMA,
        ],
    )
    return k(user_ids, movie_ids, user_emb, movie_emb)


def _mlp_kernel(u_ref, m_ref, w1u_ref, w1m_ref, b1_ref, w2_ref, b2_ref,
                w3_ref, b3_ref, out_ref):
    u = u_ref[...]
    m = m_ref[...]
    h1 = jnp.dot(u, w1u_ref[...], preferred_element_type=jnp.float32)
    h1 += jnp.dot(m, w1m_ref[...], preferred_element_type=jnp.float32)
    h1 = jnp.maximum(h1 + b1_ref[...], 0.0)
    h2 = jnp.dot(h1, w2_ref[...], preferred_element_type=jnp.float32)
    h2 = jnp.maximum(h2 + b2_ref[...], 0.0)
    logit = jnp.dot(h2, w3_ref[...], preferred_element_type=jnp.float32)
    out_ref[...] = jax.nn.sigmoid(logit + b3_ref[...])


def _tc_mlp(u_rows, m_rows, W1, b1, W2, b2, W3, b3):
    blk = 2048
    grid = (BATCH // blk,)
    w1u = W1[:, :EMB].T
    w1m = W1[:, EMB:].T
    w2 = W2.T
    w3 = W3.T
    b1r = b1.reshape(1, -1)
    b2r = b2.reshape(1, -1)
    b3r = b3.reshape(1, 1)
    out = pl.pallas_call(
        _mlp_kernel,
        grid=grid,
        in_specs=[
            pl.BlockSpec((blk, EMB), lambda i: (i, 0)),
            pl.BlockSpec((blk, EMB), lambda i: (i, 0)),
            pl.BlockSpec(w1u.shape, lambda i: (0, 0)),
            pl.BlockSpec(w1m.shape, lambda i: (0, 0)),
            pl.BlockSpec(b1r.shape, lambda i: (0, 0)),
            pl.BlockSpec(w2.shape, lambda i: (0, 0)),
            pl.BlockSpec(b2r.shape, lambda i: (0, 0)),
            pl.BlockSpec(w3.shape, lambda i: (0, 0)),
            pl.BlockSpec(b3r.shape, lambda i: (0, 0)),
        ],
        out_specs=pl.BlockSpec((blk, 1), lambda i: (i, 0)),
        out_shape=jax.ShapeDtypeStruct((BATCH, 1), jnp.float32),
    )(u_rows, m_rows, w1u, w1m, b1r, w2, b2r, w3, b3r)
    return out.reshape(BATCH)


@jax.jit
def kernel(user_ids, movie_ids, user_emb, movie_emb, W1, b1, W2, b2, W3, b3):
    u_rows, m_rows = _sc_gather(user_ids.astype(jnp.int32),
                                movie_ids.astype(jnp.int32),
                                user_emb, movie_emb)
    return _tc_mlp(u_rows, m_rows, W1, b1, W2, b2, W3, b3)
